# Initial kernel scaffold; baseline (speedup 1.0000x reference)
import jax
import jax.numpy as jnp
from jax.experimental import pallas as pl


def kernel(x, table, pos_enc):
    raise NotImplementedError("write your pallas kernel here")


# speedup vs baseline: 2.3258x; 2.3258x over previous
import functools

import jax
import jax.numpy as jnp
from jax import lax
from jax.experimental import pallas as pl
from jax.experimental.pallas import tpu as pltpu
from jax.experimental.pallas import tpu_sc as plsc

_BATCH = 4096
_SEQ = 200
_D = 64
_NW = 32
_N = _BATCH * _SEQ
_RPW = _N // _NW
_SUB = 128
_NSUB = 4
_CHUNK = _SUB * _NSUB
_NCHUNK = _RPW // _CHUNK
_LANES = 16
_VPR = _D // _LANES

_mesh = plsc.VectorSubcoreMesh(core_axis_name="c", subcore_axis_name="s")


@functools.partial(
    pl.kernel,
    out_type=jax.ShapeDtypeStruct((_N, _D), jnp.float32),
    mesh=_mesh,
    scratch_types=[
        pltpu.VMEM((_SEQ, _D), jnp.float32),
        pltpu.VMEM((_NSUB, _SUB), jnp.int32),
        pltpu.VMEM((_CHUNK, _D), jnp.float32),
        pltpu.SemaphoreType.DMA,
    ],
    compiler_params=pltpu.CompilerParams(use_tc_tiling_on_sc=False),
)
def _embed(idx_hbm, table_hbm, pos_hbm, out_hbm, pos_v, idx_v, rows_v, gsem):
    cid = lax.axis_index("c")
    sid = lax.axis_index("s")
    wid = sid * 2 + cid
    wbase = wid * _RPW

    pltpu.sync_copy(pos_hbm, pos_v)

    @pl.loop(0, _NCHUNK)
    def _chunk(g):
        base = wbase + g * _CHUNK
        for k in range(_NSUB):
            pltpu.sync_copy(idx_hbm.at[pl.ds(base + k * _SUB, _SUB)],
                            idx_v.at[k])
        copies = [
            pltpu.async_copy(table_hbm.at[idx_v.at[k]],
                             rows_v.at[pl.ds(k * _SUB, _SUB)], gsem)
            for k in range(_NSUB)
        ]
        for cp in copies:
            cp.wait()

        off = lax.rem(g * _CHUNK, _SEQ)

        @pl.loop(0, _CHUNK)
        def _row(r):
            s = lax.rem(off + r, _SEQ)
            for c in range(_VPR):
                pv = pos_v[s, pl.ds(c * _LANES, _LANES)]
                rv = rows_v[r, pl.ds(c * _LANES, _LANES)]
                rows_v[r, pl.ds(c * _LANES, _LANES)] = rv + pv

        pltpu.sync_copy(rows_v, out_hbm.at[pl.ds(base, _CHUNK)])


def kernel(x, table, pos_enc):
    idx = x.reshape(_N)
    out = _embed(idx, table, pos_enc)
    return out.reshape(_BATCH, _SEQ, _D)


# speedup vs baseline: 2.4944x; 1.0725x over previous
import functools

import jax
import jax.numpy as jnp
from jax import lax
from jax.experimental import pallas as pl
from jax.experimental.pallas import tpu as pltpu
from jax.experimental.pallas import tpu_sc as plsc

_BATCH = 4096
_SEQ = 200
_D = 64
_NW = 32
_N = _BATCH * _SEQ
_RPW = _N // _NW
_SUB = 128
_NSUB = 4
_CHUNK = _SUB * _NSUB
_NCHUNK = _RPW // _CHUNK
_NBUF = 3
_LANES = 16
_VPR = _D // _LANES

_mesh = plsc.VectorSubcoreMesh(core_axis_name="c", subcore_axis_name="s")


@functools.partial(
    pl.kernel,
    out_type=jax.ShapeDtypeStruct((_N, _D), jnp.float32),
    mesh=_mesh,
    scratch_types=[
        pltpu.VMEM((_SEQ, _D), jnp.float32),
        pltpu.VMEM((_NBUF, _NSUB, _SUB), jnp.int32),
        pltpu.VMEM((_NBUF, _CHUNK, _D), jnp.float32),
        pltpu.SemaphoreType.DMA((_NBUF,)),
        pltpu.SemaphoreType.DMA((_NBUF,)),
    ],
    compiler_params=pltpu.CompilerParams(use_tc_tiling_on_sc=False),
)
def _embed(idx_hbm, table_hbm, pos_hbm, out_hbm, pos_v, idx_v, rows_v,
           gsem, ssem):
    cid = lax.axis_index("c")
    sid = lax.axis_index("s")
    wid = sid * 2 + cid
    wbase = wid * _RPW

    def fire_gather(c, b):
        base = wbase + c * _CHUNK
        for k in range(_NSUB):
            pltpu.sync_copy(idx_hbm.at[pl.ds(base + k * _SUB, _SUB)],
                            idx_v.at[b, k])
        for k in range(_NSUB):
            pltpu.async_copy(table_hbm.at[idx_v.at[b, k]],
                             rows_v.at[b, pl.ds(k * _SUB, _SUB)],
                             gsem.at[b])

    def wait_gather(b):
        for k in range(_NSUB):
            pltpu.make_async_copy(
                table_hbm.at[idx_v.at[b, k]],
                rows_v.at[b, pl.ds(k * _SUB, _SUB)],
                gsem.at[b]).wait()

    def fire_scatter(c, b):
        base = wbase + c * _CHUNK
        pltpu.async_copy(rows_v.at[b], out_hbm.at[pl.ds(base, _CHUNK)],
                         ssem.at[b])

    def wait_scatter(b):
        pltpu.make_async_copy(rows_v.at[b], out_hbm.at[pl.ds(0, _CHUNK)],
                              ssem.at[b]).wait()

    pltpu.sync_copy(pos_hbm, pos_v)
    fire_gather(0, 0)

    @pl.loop(0, _NCHUNK)
    def _chunk(c):
        b = lax.rem(c, _NBUF)
        nb = lax.rem(c + 1, _NBUF)

        @pl.when(c + 1 < _NCHUNK)
        def _prefetch():
            @pl.when(c >= _NBUF - 1)
            def _drain():
                wait_scatter(nb)
            fire_gather(c + 1, nb)

        wait_gather(b)

        off = lax.rem(c * _CHUNK, _SEQ)

        @pl.loop(0, _CHUNK)
        def _row(r):
            s = lax.rem(off + r, _SEQ)
            for v in range(_VPR):
                pv = pos_v[s, pl.ds(v * _LANES, _LANES)]
                rv = rows_v[b, r, pl.ds(v * _LANES, _LANES)]
                rows_v[b, r, pl.ds(v * _LANES, _LANES)] = rv + pv

        fire_scatter(c, b)

    for t in range(_NBUF):
        wait_scatter(jnp.int32((_NCHUNK - 1 - t) % _NBUF))


def kernel(x, table, pos_enc):
    idx = x.reshape(_N)
    out = _embed(idx, table, pos_enc)
    return out.reshape(_BATCH, _SEQ, _D)


# speedup vs baseline: 2.5249x; 1.0122x over previous
import functools

import jax
import jax.numpy as jnp
from jax import lax
from jax.experimental import pallas as pl
from jax.experimental.pallas import tpu as pltpu
from jax.experimental.pallas import tpu_sc as plsc

_BATCH = 4096
_SEQ = 200
_D = 64
_NW = 32
_N = _BATCH * _SEQ
_RPW = _N // _NW
_SUB = 128
_NSUB = 4
_CHUNK = _SUB * _NSUB
_NCHUNK = _RPW // _CHUNK
_NBUF = 3
_LANES = 16
_VPR = _D // _LANES

_mesh = plsc.VectorSubcoreMesh(core_axis_name="c", subcore_axis_name="s")


@functools.partial(
    pl.kernel,
    out_type=jax.ShapeDtypeStruct((_N, _D), jnp.float32),
    mesh=_mesh,
    scratch_types=[
        pltpu.VMEM((_SEQ, _D), jnp.float32),
        pltpu.VMEM((_NBUF, _NSUB, _SUB), jnp.int32),
        pltpu.VMEM((_NBUF, _CHUNK, _D), jnp.float32),
        pltpu.SemaphoreType.DMA((_NBUF,)),
        pltpu.SemaphoreType.DMA((_NBUF,)),
    ],
    compiler_params=pltpu.CompilerParams(use_tc_tiling_on_sc=False),
)
def _embed(idx_hbm, table_hbm, pos_hbm, out_hbm, pos_v, idx_v, rows_v,
           gsem, ssem):
    cid = lax.axis_index("c")
    sid = lax.axis_index("s")
    wid = sid * 2 + cid
    wbase = wid * _RPW

    def fire_gather(c, b):
        base = wbase + c * _CHUNK
        for k in range(_NSUB):
            pltpu.sync_copy(idx_hbm.at[pl.ds(base + k * _SUB, _SUB)],
                            idx_v.at[b, k])
        for k in range(_NSUB):
            pltpu.async_copy(table_hbm.at[idx_v.at[b, k]],
                             rows_v.at[b, pl.ds(k * _SUB, _SUB)],
                             gsem.at[b])

    def wait_gather(b):
        for k in range(_NSUB):
            pltpu.make_async_copy(
                table_hbm.at[idx_v.at[b, k]],
                rows_v.at[b, pl.ds(k * _SUB, _SUB)],
                gsem.at[b]).wait()

    def fire_scatter(c, b):
        base = wbase + c * _CHUNK
        pltpu.async_copy(rows_v.at[b], out_hbm.at[pl.ds(base, _CHUNK)],
                         ssem.at[b])

    def wait_scatter(b):
        pltpu.make_async_copy(rows_v.at[b], out_hbm.at[pl.ds(0, _CHUNK)],
                              ssem.at[b]).wait()

    pltpu.sync_copy(pos_hbm, pos_v)
    fire_gather(0, 0)

    @pl.loop(0, _NCHUNK)
    def _chunk(c):
        b = lax.rem(c, _NBUF)
        nb = lax.rem(c + 1, _NBUF)

        @pl.when(c + 1 < _NCHUNK)
        def _prefetch():
            @pl.when(c >= _NBUF - 1)
            def _drain():
                wait_scatter(nb)
            fire_gather(c + 1, nb)

        wait_gather(b)

        off = lax.rem(c * _CHUNK, _SEQ)

        @pl.loop(0, _CHUNK, unroll=8)
        def _row(r):
            s = lax.rem(off + r, _SEQ)
            for v in range(_VPR):
                pv = pos_v[s, pl.ds(v * _LANES, _LANES)]
                rv = rows_v[b, r, pl.ds(v * _LANES, _LANES)]
                rows_v[b, r, pl.ds(v * _LANES, _LANES)] = rv + pv

        fire_scatter(c, b)

    for t in range(_NBUF):
        wait_scatter(jnp.int32((_NCHUNK - 1 - t) % _NBUF))


def kernel(x, table, pos_enc):
    idx = x.reshape(_N)
    out = _embed(idx, table, pos_enc)
    return out.reshape(_BATCH, _SEQ, _D)


# speedup vs baseline: 2.9380x; 1.1636x over previous
import functools

import jax
import jax.numpy as jnp
from jax import lax
from jax.experimental import pallas as pl
from jax.experimental.pallas import tpu as pltpu
from jax.experimental.pallas import tpu_sc as plsc

_BATCH = 4096
_SEQ = 200
_D = 64
_NW = 32
_N = _BATCH * _SEQ
_RPW = _N // _NW
_SUB = 128
_NSUB = 4
_CHUNK = _SUB * _NSUB
_NCHUNK = _RPW // _CHUNK
_NBUF = 3
_LANES = 16
_VPR = _D // _LANES

_mesh = plsc.VectorSubcoreMesh(core_axis_name="c", subcore_axis_name="s")


@functools.partial(
    pl.kernel,
    out_type=jax.ShapeDtypeStruct((_N, _D), jnp.float32),
    mesh=_mesh,
    scratch_types=[
        pltpu.VMEM((_SEQ, _D), jnp.float32),
        pltpu.VMEM((_NBUF, _NSUB, _SUB), jnp.int32),
        pltpu.VMEM((_NBUF, _CHUNK, _D), jnp.float32),
        pltpu.SemaphoreType.DMA((_NBUF,)),
        pltpu.SemaphoreType.DMA((_NBUF,)),
    ],
    compiler_params=pltpu.CompilerParams(use_tc_tiling_on_sc=False),
)
def _embed(idx_hbm, table_hbm, pos_hbm, out_hbm, pos_v, idx_v, rows_v,
           gsem, ssem):
    cid = lax.axis_index("c")
    sid = lax.axis_index("s")
    wid = sid * 2 + cid
    wbase = wid * _RPW

    def fire_gather(c, b):
        base = wbase + c * _CHUNK
        for k in range(_NSUB):
            pltpu.sync_copy(idx_hbm.at[pl.ds(base + k * _SUB, _SUB)],
                            idx_v.at[b, k])
        for k in range(_NSUB):
            pltpu.async_copy(table_hbm.at[idx_v.at[b, k]],
                             rows_v.at[b, pl.ds(k * _SUB, _SUB)],
                             gsem.at[b])

    def wait_gather(b):
        for k in range(_NSUB):
            pltpu.make_async_copy(
                table_hbm.at[idx_v.at[b, k]],
                rows_v.at[b, pl.ds(k * _SUB, _SUB)],
                gsem.at[b]).wait()

    def fire_scatter(c, b):
        base = wbase + c * _CHUNK
        pltpu.async_copy(rows_v.at[b], out_hbm.at[pl.ds(base, _CHUNK)],
                         ssem.at[b])

    def wait_scatter(b):
        pltpu.make_async_copy(rows_v.at[b], out_hbm.at[pl.ds(0, _CHUNK)],
                              ssem.at[b]).wait()

    pltpu.sync_copy(pos_hbm, pos_v)
    fire_gather(0, 0)

    @pl.loop(0, _NCHUNK)
    def _chunk(c):
        b = lax.rem(c, _NBUF)
        nb = lax.rem(c + 1, _NBUF)

        @pl.when(c + 1 < _NCHUNK)
        def _prefetch():
            @pl.when(c >= _NBUF - 1)
            def _drain():
                wait_scatter(nb)
            fire_gather(c + 1, nb)

        wait_gather(b)

        off = lax.rem(c * _CHUNK, _SEQ)

        @pl.loop(0, _CHUNK, unroll=8)
        def _row(r):
            s = lax.rem(off + r, _SEQ)
            for v in range(_VPR):
                pv = pos_v[s, pl.ds(v * _LANES, _LANES)]
                plsc.addupdate(rows_v.at[b, r, pl.ds(v * _LANES, _LANES)], pv)

        fire_scatter(c, b)

    for t in range(_NBUF):
        wait_scatter(jnp.int32((_NCHUNK - 1 - t) % _NBUF))


def kernel(x, table, pos_enc):
    idx = x.reshape(_N)
    out = _embed(idx, table, pos_enc)
    return out.reshape(_BATCH, _SEQ, _D)


# speedup vs baseline: 2.9504x; 1.0042x over previous
import functools

import jax
import jax.numpy as jnp
from jax import lax
from jax.experimental import pallas as pl
from jax.experimental.pallas import tpu as pltpu
from jax.experimental.pallas import tpu_sc as plsc

_BATCH = 4096
_SEQ = 200
_D = 64
_NW = 32
_N = _BATCH * _SEQ
_RPW = _N // _NW
_SUB = 128
_NSUB = 2
_CHUNK = _SUB * _NSUB
_NCHUNK = _RPW // _CHUNK
_NBUF = 4
_LOOKAHEAD = 2
_LANES = 16
_VPR = _D // _LANES

_mesh = plsc.VectorSubcoreMesh(core_axis_name="c", subcore_axis_name="s")


@functools.partial(
    pl.kernel,
    out_type=jax.ShapeDtypeStruct((_N, _D), jnp.float32),
    mesh=_mesh,
    scratch_types=[
        pltpu.VMEM((_SEQ, _D), jnp.float32),
        pltpu.VMEM((_NBUF, _NSUB, _SUB), jnp.int32),
        pltpu.VMEM((_NBUF, _CHUNK, _D), jnp.float32),
        pltpu.SemaphoreType.DMA((_NBUF,)),
        pltpu.SemaphoreType.DMA((_NBUF,)),
    ],
    compiler_params=pltpu.CompilerParams(use_tc_tiling_on_sc=False),
)
def _embed(idx_hbm, table_hbm, pos_hbm, out_hbm, pos_v, idx_v, rows_v,
           gsem, ssem):
    cid = lax.axis_index("c")
    sid = lax.axis_index("s")
    wid = sid * 2 + cid
    wbase = wid * _RPW

    def fire_gather(c, b):
        base = wbase + c * _CHUNK
        for k in range(_NSUB):
            pltpu.sync_copy(idx_hbm.at[pl.ds(base + k * _SUB, _SUB)],
                            idx_v.at[b, k])
        for k in range(_NSUB):
            pltpu.async_copy(table_hbm.at[idx_v.at[b, k]],
                             rows_v.at[b, pl.ds(k * _SUB, _SUB)],
                             gsem.at[b])

    def wait_gather(b):
        for k in range(_NSUB):
            pltpu.make_async_copy(
                table_hbm.at[idx_v.at[b, k]],
                rows_v.at[b, pl.ds(k * _SUB, _SUB)],
                gsem.at[b]).wait()

    def fire_scatter(c, b):
        base = wbase + c * _CHUNK
        pltpu.async_copy(rows_v.at[b], out_hbm.at[pl.ds(base, _CHUNK)],
                         ssem.at[b])

    def wait_scatter(b):
        pltpu.make_async_copy(rows_v.at[b], out_hbm.at[pl.ds(0, _CHUNK)],
                              ssem.at[b]).wait()

    pltpu.sync_copy(pos_hbm, pos_v)
    for p in range(_LOOKAHEAD):
        fire_gather(p, p)

    @pl.loop(0, _NCHUNK)
    def _chunk(c):
        b = lax.rem(c, _NBUF)

        @pl.when(c + _LOOKAHEAD < _NCHUNK)
        def _prefetch():
            nb = lax.rem(c + _LOOKAHEAD, _NBUF)

            @pl.when(c >= _NBUF - _LOOKAHEAD)
            def _drain():
                wait_scatter(nb)
            fire_gather(c + _LOOKAHEAD, nb)

        wait_gather(b)

        off = lax.rem(c * _CHUNK, _SEQ)

        @pl.loop(0, _CHUNK, unroll=8)
        def _row(r):
            s = lax.rem(off + r, _SEQ)
            for v in range(_VPR):
                pv = pos_v[s, pl.ds(v * _LANES, _LANES)]
                plsc.addupdate(rows_v.at[b, r, pl.ds(v * _LANES, _LANES)], pv)

        fire_scatter(c, b)

    for t in range(_NBUF):
        wait_scatter(jnp.int32((_NCHUNK - 1 - t) % _NBUF))


def kernel(x, table, pos_enc):
    idx = x.reshape(_N)
    out = _embed(idx, table, pos_enc)
    return out.reshape(_BATCH, _SEQ, _D)
